# baseline (speedup 1.0000x reference)
import functools

import jax
import jax.numpy as jnp
from jax.experimental import pallas as pl


def _mlp_kernel(x_ref, w1_ref, b1_ref, w2_ref, b2_ref, o_ref):
    x = x_ref[...].astype(jnp.bfloat16)
    h = jnp.dot(x, w1_ref[...].astype(jnp.bfloat16),
                preferred_element_type=jnp.float32)
    h = jnp.maximum(h + b1_ref[...], 0.0).astype(jnp.bfloat16)
    o = jnp.dot(h, w2_ref[...].astype(jnp.bfloat16),
                preferred_element_type=jnp.float32)
    o_ref[...] = o + b2_ref[...]


@functools.partial(jax.jit, static_argnames=("block_b",))
def _fused_mlp(x, W1, b1, W2, b2, block_b):
    batch, n_in = x.shape
    n_mid = W1.shape[1]
    n_out = W2.shape[1]
    grid = (batch // block_b,)
    return pl.pallas_call(
        _mlp_kernel,
        grid=grid,
        in_specs=[
            pl.BlockSpec((block_b, n_in), lambda i: (i, 0)),
            pl.BlockSpec((n_in, n_mid), lambda i: (0, 0)),
            pl.BlockSpec((1, n_mid), lambda i: (0, 0)),
            pl.BlockSpec((n_mid, n_out), lambda i: (0, 0)),
            pl.BlockSpec((1, n_out), lambda i: (0, 0)),
        ],
        out_specs=pl.BlockSpec((block_b, n_out), lambda i: (i, 0)),
        out_shape=jax.ShapeDtypeStruct((batch, n_out), jnp.float32),
        compiler_params=pltpu_params(),
    )(x, W1, b1.reshape(1, n_mid), W2, b2.reshape(1, n_out))


def pltpu_params():
    from jax.experimental.pallas import tpu as pltpu

    return pltpu.CompilerParams(
        dimension_semantics=("arbitrary",),
    )


def kernel(x, W1, b1, W2, b2):
    return _fused_mlp(x, W1, b1, W2, b2, block_b=1024)


# speedup vs baseline: 1.6847x; 1.6847x over previous
import functools

import jax
import jax.numpy as jnp
from jax.experimental import pallas as pl


def _mlp_kernel(x_ref, w1_ref, b1_ref, w2_ref, b2_ref, o_ref):
    x = x_ref[...].astype(jnp.bfloat16)
    h = jnp.dot(x, w1_ref[...].astype(jnp.bfloat16),
                preferred_element_type=jnp.float32)
    h = jnp.maximum(h + b1_ref[...], 0.0).astype(jnp.bfloat16)
    o = jnp.dot(h, w2_ref[...].astype(jnp.bfloat16),
                preferred_element_type=jnp.float32)
    o_ref[...] = o + b2_ref[...]


@functools.partial(jax.jit, static_argnames=("block_b",))
def _fused_mlp(x, W1, b1, W2, b2, block_b):
    batch, n_in = x.shape
    n_mid = W1.shape[1]
    n_out = W2.shape[1]
    grid = (batch // block_b,)
    return pl.pallas_call(
        _mlp_kernel,
        grid=grid,
        in_specs=[
            pl.BlockSpec((block_b, n_in), lambda i: (i, 0)),
            pl.BlockSpec((n_in, n_mid), lambda i: (0, 0)),
            pl.BlockSpec((1, n_mid), lambda i: (0, 0)),
            pl.BlockSpec((n_mid, n_out), lambda i: (0, 0)),
            pl.BlockSpec((1, n_out), lambda i: (0, 0)),
        ],
        out_specs=pl.BlockSpec((block_b, n_out), lambda i: (i, 0)),
        out_shape=jax.ShapeDtypeStruct((batch, n_out), jnp.float32),
        compiler_params=pltpu_params(),
    )(x, W1, b1.reshape(1, n_mid), W2, b2.reshape(1, n_out))


def pltpu_params():
    from jax.experimental.pallas import tpu as pltpu

    return pltpu.CompilerParams(
        dimension_semantics=("arbitrary",),
    )


def kernel(x, W1, b1, W2, b2):
    return _fused_mlp(x, W1, b1, W2, b2, block_b=4096)


# speedup vs baseline: 1.7467x; 1.0368x over previous
import functools

import jax
import jax.numpy as jnp
from jax.experimental import pallas as pl


def _mlp_kernel(x_ref, w1_ref, b1_ref, w2_ref, b2_ref, o_ref):
    x = x_ref[...].astype(jnp.bfloat16)
    h = jnp.dot(x, w1_ref[...].astype(jnp.bfloat16),
                preferred_element_type=jnp.float32)
    h = jnp.maximum(h + b1_ref[...], 0.0).astype(jnp.bfloat16)
    o = jnp.dot(h, w2_ref[...].astype(jnp.bfloat16),
                preferred_element_type=jnp.float32)
    o_ref[...] = o + b2_ref[...]


@functools.partial(jax.jit, static_argnames=("block_b",))
def _fused_mlp(x, W1, b1, W2, b2, block_b):
    batch, n_in = x.shape
    n_mid = W1.shape[1]
    n_out = W2.shape[1]
    grid = (batch // block_b,)
    return pl.pallas_call(
        _mlp_kernel,
        grid=grid,
        in_specs=[
            pl.BlockSpec((block_b, n_in), lambda i: (i, 0)),
            pl.BlockSpec((n_in, n_mid), lambda i: (0, 0)),
            pl.BlockSpec((1, n_mid), lambda i: (0, 0)),
            pl.BlockSpec((n_mid, n_out), lambda i: (0, 0)),
            pl.BlockSpec((1, n_out), lambda i: (0, 0)),
        ],
        out_specs=pl.BlockSpec((block_b, n_out), lambda i: (i, 0)),
        out_shape=jax.ShapeDtypeStruct((batch, n_out), jnp.float32),
        compiler_params=pltpu_params(),
    )(x, W1, b1.reshape(1, n_mid), W2, b2.reshape(1, n_out))


def pltpu_params():
    from jax.experimental.pallas import tpu as pltpu

    return pltpu.CompilerParams(
        dimension_semantics=("arbitrary",),
    )


def kernel(x, W1, b1, W2, b2):
    return _fused_mlp(x, W1, b1, W2, b2, block_b=8192)
